# baseline (speedup 1.0000x reference)
import functools

import jax
import jax.numpy as jnp
from jax.experimental import pallas as pl

NUM_EMBEDDINGS = 1024
EMBEDDING_DIM = 64
TOKENS = 16 * 32 * 32
BLOCK_TOKENS = 1024
NUM_BLOCKS = TOKENS // BLOCK_TOKENS


def _vq_block(z_ref, emb_ref, e_sq_ref, quant_ref, idx_ref):
    z = z_ref[...]
    emb = emb_ref[...]
    e_sq = e_sq_ref[...]
    z_sq = jnp.sum(z * z, axis=1, keepdims=True)
    dot = jax.lax.dot_general(
        z, emb, (((1,), (1,)), ((), ())),
        preferred_element_type=jnp.float32)
    dist = z_sq + e_sq - 2.0 * dot
    minv = jnp.min(dist, axis=1, keepdims=True)
    iota = jax.lax.broadcasted_iota(jnp.int32, dist.shape, 1)
    idx = jnp.min(jnp.where(dist == minv, iota, NUM_EMBEDDINGS),
                  axis=1).astype(jnp.int32)
    onehot = (idx[:, None] ==
              jax.lax.broadcasted_iota(jnp.int32, (1, NUM_EMBEDDINGS), 1)
              ).astype(jnp.float32)
    quant = jax.lax.dot_general(
        onehot, emb, (((1,), (0,)), ((), ())),
        preferred_element_type=jnp.float32)
    quant_ref[...] = quant
    idx_ref[0, 0, :] = idx


@functools.partial(jax.jit, static_argnums=())
def kernel(hidden_states, embedding):
    flat = hidden_states.reshape(TOKENS, EMBEDDING_DIM)
    e_sq = jnp.sum(embedding * embedding, axis=1)[None, :]

    quant, idx = pl.pallas_call(
        _vq_block,
        grid=(NUM_BLOCKS,),
        in_specs=[
            pl.BlockSpec((BLOCK_TOKENS, EMBEDDING_DIM), lambda b: (b, 0)),
            pl.BlockSpec((NUM_EMBEDDINGS, EMBEDDING_DIM), lambda b: (0, 0)),
            pl.BlockSpec((1, NUM_EMBEDDINGS), lambda b: (0, 0)),
        ],
        out_specs=[
            pl.BlockSpec((BLOCK_TOKENS, EMBEDDING_DIM), lambda b: (b, 0)),
            pl.BlockSpec((1, 1, BLOCK_TOKENS), lambda b: (b, 0, 0)),
        ],
        out_shape=[
            jax.ShapeDtypeStruct((TOKENS, EMBEDDING_DIM), jnp.float32),
            jax.ShapeDtypeStruct((NUM_BLOCKS, 1, BLOCK_TOKENS), jnp.int32),
        ],
    )(flat, embedding, e_sq)

    z_q = quant.reshape(hidden_states.shape)
    B = hidden_states.shape[0]
    min_encoding_indices = idx.reshape(B, TOKENS // B)
    return (z_q, min_encoding_indices)
